# baseline (speedup 1.0000x reference)
import functools

import jax
import jax.numpy as jnp
import numpy as np
from jax import lax
from jax.experimental import pallas as pl
from jax.experimental.pallas import tpu as pltpu
from jax.experimental.pallas import tpu_sc as plsc

N = 50000
E = 800000
FD = 2
SD = 32
HD = 32
PSTEP = 4

NP_ = 50176
EP_ = 819200
TW = 48
MF = 16
CHUNK = 128
NWORK = 32
CPW = EP_ // (NWORK * CHUNK)
STRIPE = NP_ // 16

BE = 2048
BN = 1024

@functools.lru_cache(maxsize=None)
def _build_gather():
    mesh = plsc.VectorSubcoreMesh(core_axis_name="c", subcore_axis_name="s")

    @functools.partial(
        pl.kernel,
        out_type=(
            jax.ShapeDtypeStruct((EP_, TW), jnp.float32),
            jax.ShapeDtypeStruct((EP_, TW), jnp.float32),
        ),
        scratch_types=[
            pltpu.VMEM((CHUNK,), jnp.int32),
            pltpu.VMEM((CHUNK, TW), jnp.float32),
            pltpu.SemaphoreType.DMA,
        ],
        mesh=mesh,
        compiler_params=pltpu.CompilerParams(use_tc_tiling_on_sc=False),
    )
    def _gather_k(tab, srcp, dstp, gsrc, gdst, idxbuf, rowbuf, sem):
        wid = lax.axis_index("s") * 2 + lax.axis_index("c")

        def chunk(t, _):
            base = (wid * CPW + t) * CHUNK
            pltpu.sync_copy(srcp.at[pl.ds(base, CHUNK)], idxbuf)
            pltpu.async_copy(tab.at[idxbuf], rowbuf, sem).wait()
            pltpu.sync_copy(rowbuf, gsrc.at[pl.ds(base, CHUNK)])
            pltpu.sync_copy(dstp.at[pl.ds(base, CHUNK)], idxbuf)
            pltpu.async_copy(tab.at[idxbuf], rowbuf, sem).wait()
            pltpu.sync_copy(rowbuf, gdst.at[pl.ds(base, CHUNK)])
            return _

        lax.fori_loop(0, CPW, chunk, None)

    return _gather_k


@functools.lru_cache(maxsize=None)
def _build_scatter(w):
    mesh = plsc.VectorSubcoreMesh(core_axis_name="c", subcore_axis_name="s")

    @functools.partial(
        pl.kernel,
        out_type=jax.ShapeDtypeStruct((2 * NP_, w), jnp.float32),
        scratch_types=[
            pltpu.VMEM((CHUNK,), jnp.int32),
            pltpu.VMEM((CHUNK, w), jnp.float32),
            pltpu.VMEM_SHARED((NP_, w), jnp.float32),
            pltpu.SemaphoreType.DMA,
        ],
        mesh=mesh,
        compiler_params=pltpu.CompilerParams(use_tc_tiling_on_sc=False),
    )
    def _scatter_k(msg, idx, zrows, part, idxbuf, rowbuf, accum, sem):
        c = lax.axis_index("c")
        s_ = lax.axis_index("s")
        wid = s_ * 2 + c
        sbase = s_ * STRIPE
        pltpu.sync_copy(zrows.at[pl.ds(sbase, STRIPE)],
                        accum.at[pl.ds(sbase, STRIPE)])
        plsc.subcore_barrier()

        def chunk(t, _):
            base = (wid * CPW + t) * CHUNK
            pltpu.sync_copy(idx.at[pl.ds(base, CHUNK)], idxbuf)
            pltpu.sync_copy(msg.at[pl.ds(base, CHUNK)], rowbuf)
            pltpu.sync_copy(rowbuf, accum.at[idxbuf], add=True)
            return _

        lax.fori_loop(0, CPW, chunk, None)
        plsc.subcore_barrier()
        pltpu.sync_copy(accum.at[pl.ds(sbase, STRIPE)],
                        part.at[pl.ds(c * NP_ + sbase, STRIPE)])

    return _scatter_k


def _fcol(a, q):
    if q < 2:
        return ("gs", 2 * a + q)
    if q < 4:
        return ("gd", 2 * a + q - 2)
    return ("ef", a)


def _tcol(a, q):
    if q < 2:
        return ("tab", 2 * a + q)
    return ("fc", 2 * a + q - 2)


def _sel(shapes, entries):
    mats = {k: np.zeros(v, np.float32) for k, v in shapes.items()}
    for (src, row), col in entries:
        mats[src][row, col] = 1.0
    return mats


def _edge_consts():
    shapes = {"gs": (TW, 75), "gd": (TW, 75), "ef": (3, 75)}
    EA = _sel(shapes, [(_fcol(j, i), j * 25 + i * 5 + k)
                       for j in range(3) for i in range(5) for k in range(5)])
    EB = _sel(shapes, [(_fcol(j, k), j * 25 + i * 5 + k)
                       for j in range(3) for i in range(5) for k in range(5)])
    EC = np.zeros((75, 25), np.float32)
    for j in range(3):
        for i in range(5):
            for k in range(5):
                EC[j * 25 + i * 5 + k, i * 5 + k] = 1.0
    sh2 = {"gs": (TW, 30), "gd": (TW, 30), "ef": (3, 30)}
    MA = _sel(sh2, [(_fcol(i, j), i * 10 + k * 5 + j)
                    for i in range(3) for k in range(2) for j in range(5)])
    MB = np.zeros((42, 30), np.float32)
    MC = np.zeros((30, MF), np.float32)
    for i in range(3):
        for k in range(2):
            for j in range(5):
                MB[2 * j + k, i * 10 + k * 5 + j] = 1.0
                MC[i * 10 + k * 5 + j, i * 2 + k] = 1.0
    MS = np.zeros((42, SD), np.float32)
    for m in range(SD):
        MS[10 + m, m] = 1.0
    CNT = np.zeros((1, MF), np.float32)
    CNT[0, 6] = 1.0
    return (EA["gs"], EA["gd"], EA["ef"], EB["gs"], EB["gd"], EB["ef"], EC,
            MA["gs"], MA["gd"], MA["ef"], MB, MC, MS, CNT)


def _node_consts():
    shapes = {"tab": (TW, 48), "fc": (MF, 48)}
    NA = _sel(shapes, [(_tcol(j, i), j * 16 + i * 4 + k)
                       for j in range(3) for i in range(4) for k in range(4)])
    NB = _sel(shapes, [(_tcol(j, k), j * 16 + i * 4 + k)
                       for j in range(3) for i in range(4) for k in range(4)])
    NC = np.zeros((48, 16), np.float32)
    for j in range(3):
        for i in range(4):
            for k in range(4):
                NC[j * 16 + i * 4 + k, i * 4 + k] = 1.0
    sh2 = {"tab": (TW, 24), "fc": (MF, 24)}
    PA = _sel(sh2, [(_tcol(i, j), i * 8 + k * 4 + j)
                    for i in range(3) for k in range(2) for j in range(4)])
    Q = np.zeros((40, 24), np.float32)
    R = np.zeros((24, TW), np.float32)
    for i in range(3):
        for k in range(2):
            for j in range(4):
                Q[2 * j + k, i * 8 + k * 4 + j] = 1.0
                R[i * 8 + k * 4 + j, i * 2 + k] = 1.0
    S = np.zeros((40, TW), np.float32)
    for m in range(SD):
        S[8 + m, 16 + m] = 1.0
    E6 = np.zeros((MF, 1), np.float32)
    E6[6, 0] = 1.0
    return (NA["tab"], NA["fc"], NB["tab"], NB["fc"], NC,
            PA["tab"], PA["fc"], Q, R, S, E6)


_EDGE_C = _edge_consts()
_NODE_C = _node_consts()


def _silu(x):
    return jax.nn.silu(x)


def _mm(a, b):
    return jnp.dot(a, b, preferred_element_type=jnp.float32,
                   precision=lax.Precision.HIGHEST)


def _mmd(a, b):
    return jnp.dot(a, b, preferred_element_type=jnp.float32)


def _edge_body(gs_ref, gd_ref, ef_ref, es_ref,
               w1g, w1gs, w1gd, w1es, b1, w2, b2, w3, b3,
               ea_gs, ea_gd, ea_ef, eb_gs, eb_gd, eb_ef, ec,
               ma_gs, ma_gd, ma_ef, mb, mc, ms, cnt,
               msgf_ref, msgs_ref):
    gs = gs_ref[...]
    gd = gd_ref[...]
    ef = ef_ref[...]
    ag = _mm(gs, ea_gs[...]) + _mm(gd, ea_gd[...]) + _mm(ef, ea_ef[...])
    bg = _mm(gs, eb_gs[...]) + _mm(gd, eb_gd[...]) + _mm(ef, eb_ef[...])
    gram = _mm(ag * bg, ec[...])
    ss = jnp.sum(gram * gram, axis=1, keepdims=True)
    gram = gram / jnp.maximum(jnp.sqrt(ss), 1e-12)
    h = _silu(_mmd(gram, w1g[...]) + _mmd(gs, w1gs[...])
              + _mmd(gd, w1gd[...]) + _mmd(es_ref[...], w1es[...]) + b1[...])
    h = _silu(_mmd(h, w2[...]) + b2[...])
    cc = _mmd(h, w3[...]) + b3[...]
    am = _mm(gs, ma_gs[...]) + _mm(gd, ma_gd[...]) + _mm(ef, ma_ef[...])
    bm = _mm(cc, mb[...])
    msgf_ref[...] = _mm(am * bm, mc[...]) + cnt[...]
    msgs_ref[...] = _mm(cc, ms[...])


def _edge_call(gsrc, gdst, efp, esp, nw, ecst):
    nb = EP_ // BE
    full = lambda a: pl.BlockSpec(a.shape, lambda i: (0,) * a.ndim)
    return pl.pallas_call(
        _edge_body,
        grid=(nb,),
        in_specs=[
            pl.BlockSpec((BE, TW), lambda i: (i, 0)),
            pl.BlockSpec((BE, TW), lambda i: (i, 0)),
            pl.BlockSpec((BE, 3), lambda i: (i, 0)),
            pl.BlockSpec((BE, 4), lambda i: (i, 0)),
        ] + [full(a) for a in nw] + [full(a) for a in ecst],
        out_specs=[
            pl.BlockSpec((BE, MF), lambda i: (i, 0)),
            pl.BlockSpec((BE, SD), lambda i: (i, 0)),
        ],
        out_shape=[
            jax.ShapeDtypeStruct((EP_, MF), jnp.float32),
            jax.ShapeDtypeStruct((EP_, SD), jnp.float32),
        ],
    )(gsrc, gdst, efp, esp, *nw, *ecst)


def _node_body(tab_ref, fp0, fp1, sp0, sp1,
               w1g, w1tab, w1sc, b1, w2, b2, w3, b3,
               na_tab, na_fc, nb_tab, nb_fc, nc,
               pa_tab, pa_fc, q, r, s_, e6, out_ref):
    tab = tab_ref[...]
    fsum = fp0[...] + fp1[...]
    inv = 1.0 / jnp.maximum(_mm(fsum, e6[...]), 1.0)
    fci = fsum * inv
    ssum = (sp0[...] + sp1[...]) * inv
    ag = _mm(tab, na_tab[...]) + _mm(fci, na_fc[...])
    bg = _mm(tab, nb_tab[...]) + _mm(fci, nb_fc[...])
    gram = _mm(ag * bg, nc[...])
    ss = jnp.sum(gram * gram, axis=1, keepdims=True)
    gram = gram / jnp.maximum(jnp.sqrt(ss), 1e-12)
    h = _silu(_mmd(gram, w1g[...]) + _mmd(tab, w1tab[...])
              + _mmd(ssum, w1sc[...]) + b1[...])
    h = _silu(_mmd(h, w2[...]) + b2[...])
    tc = _mmd(h, w3[...]) + b3[...]
    a2 = _mm(tab, pa_tab[...]) + _mm(fci, pa_fc[...])
    b2_ = _mm(tc, q[...])
    out_ref[...] = _mm(a2 * b2_, r[...]) + _mm(tc, s_[...])


def _node_call(tab, fpart, spart, sw, ncst):
    nb = NP_ // BN
    off = NP_ // BN
    full = lambda a: pl.BlockSpec(a.shape, lambda i: (0,) * a.ndim)
    return pl.pallas_call(
        _node_body,
        grid=(nb,),
        in_specs=[
            pl.BlockSpec((BN, TW), lambda i: (i, 0)),
            pl.BlockSpec((BN, MF), lambda i: (i, 0)),
            pl.BlockSpec((BN, MF), lambda i: (i + off, 0)),
            pl.BlockSpec((BN, SD), lambda i: (i, 0)),
            pl.BlockSpec((BN, SD), lambda i: (i + off, 0)),
        ] + [full(a) for a in sw] + [full(a) for a in ncst],
        out_specs=pl.BlockSpec((BN, TW), lambda i: (i, 0)),
        out_shape=jax.ShapeDtypeStruct((NP_, TW), jnp.float32),
    )(tab, fpart, fpart, spart, spart, *sw, *ncst)


def _edge_weights(p):
    w1 = p["W1"]
    return (w1[:25],
            jnp.pad(w1[25:57], ((16, 0), (0, 0))),
            jnp.pad(w1[57:89], ((16, 0), (0, 0))),
            w1[89:93],
            p["b1"].reshape(1, -1), p["W2"], p["b2"].reshape(1, -1),
            p["W3"], p["b3"].reshape(1, -1))


def _node_weights(p):
    w1 = p["W1"]
    return (w1[:16],
            jnp.pad(w1[16:48], ((16, 0), (0, 0))),
            w1[48:80],
            p["b1"].reshape(1, -1), p["W2"], p["b2"].reshape(1, -1),
            p["W3"], p["b3"].reshape(1, -1))


def kernel(f, s, edge_index, edge_f, edge_s, net, self_net):
    ei = edge_index.astype(jnp.int32)
    pad = jnp.full((EP_ - E,), N, jnp.int32)
    srcp = jnp.concatenate([ei[0], pad])
    dstp = jnp.concatenate([ei[1], pad])
    efp = jnp.pad(edge_f.reshape(E, 3), ((0, EP_ - E), (0, 0)))
    esp = jnp.pad(edge_s, ((0, EP_ - E), (0, 0)))
    tab = jnp.concatenate([
        jnp.pad(f.reshape(N, 6), ((0, NP_ - N), (0, 0))),
        jnp.zeros((NP_, 10), jnp.float32),
        jnp.pad(s, ((0, NP_ - N), (0, 0))),
    ], axis=1)
    zf = jnp.zeros((NP_, MF), jnp.float32)
    zs = jnp.zeros((NP_, SD), jnp.float32)
    nw = _edge_weights(net)
    sw = _node_weights(self_net)
    ecst = tuple(jnp.asarray(m) for m in _EDGE_C)
    ncst = tuple(jnp.asarray(m) for m in _NODE_C)
    gather_k = _build_gather()
    scatter_f = _build_scatter(MF)
    scatter_s = _build_scatter(SD)
    for _ in range(PSTEP):
        gsrc, gdst = gather_k(tab, srcp, dstp)
        msgf, msgs = _edge_call(gsrc, gdst, efp, esp, nw, ecst)
        fpart = scatter_f(msgf, srcp, zf)
        spart = scatter_s(msgs, srcp, zs)
        tab = _node_call(tab, fpart, spart, sw, ncst)
    return tab[:N, :6].reshape(N, 3, FD), tab[:N, 16:48]


# speedup vs baseline: 1.8191x; 1.8191x over previous
import functools

import jax
import jax.numpy as jnp
import numpy as np
from jax import lax
from jax.experimental import pallas as pl
from jax.experimental.pallas import tpu as pltpu
from jax.experimental.pallas import tpu_sc as plsc

N = 50000
E = 800000
FD = 2
SD = 32
HD = 32
PSTEP = 4

NP_ = 50176
EP_ = 819200
TW = 48
MF = 16
CHUNK = 128
NWORK = 32
CPW = EP_ // (NWORK * CHUNK)
STRIPE = NP_ // 16

BE = 2048
BN = 1024

@functools.lru_cache(maxsize=None)
def _build_gather():
    mesh = plsc.VectorSubcoreMesh(core_axis_name="c", subcore_axis_name="s")

    @functools.partial(
        pl.kernel,
        out_type=(
            jax.ShapeDtypeStruct((EP_, TW), jnp.float32),
            jax.ShapeDtypeStruct((EP_, TW), jnp.float32),
        ),
        scratch_types=[
            pltpu.VMEM((CHUNK,), jnp.int32),
            pltpu.VMEM((CHUNK, TW), jnp.float32),
            pltpu.SemaphoreType.DMA,
        ],
        mesh=mesh,
        compiler_params=pltpu.CompilerParams(use_tc_tiling_on_sc=False),
    )
    def _gather_k(tab, srcp, dstp, gsrc, gdst, idxbuf, rowbuf, sem):
        wid = lax.axis_index("s") * 2 + lax.axis_index("c")

        def chunk(t, _):
            base = (wid * CPW + t) * CHUNK
            pltpu.sync_copy(srcp.at[pl.ds(base, CHUNK)], idxbuf)
            pltpu.async_copy(tab.at[idxbuf], rowbuf, sem).wait()
            pltpu.sync_copy(rowbuf, gsrc.at[pl.ds(base, CHUNK)])
            pltpu.sync_copy(dstp.at[pl.ds(base, CHUNK)], idxbuf)
            pltpu.async_copy(tab.at[idxbuf], rowbuf, sem).wait()
            pltpu.sync_copy(rowbuf, gdst.at[pl.ds(base, CHUNK)])
            return _

        lax.fori_loop(0, CPW, chunk, None)

    return _gather_k


@functools.lru_cache(maxsize=None)
def _build_scatter(w):
    mesh = plsc.VectorSubcoreMesh(core_axis_name="c", subcore_axis_name="s")

    @functools.partial(
        pl.kernel,
        out_type=jax.ShapeDtypeStruct((2 * NP_, w), jnp.float32),
        scratch_types=[
            pltpu.VMEM((CHUNK,), jnp.int32),
            pltpu.VMEM((CHUNK, w), jnp.float32),
            pltpu.VMEM_SHARED((NP_, w), jnp.float32),
            pltpu.SemaphoreType.DMA,
        ],
        mesh=mesh,
        compiler_params=pltpu.CompilerParams(use_tc_tiling_on_sc=False),
    )
    def _scatter_k(msg, idx, zrows, part, idxbuf, rowbuf, accum, sem):
        c = lax.axis_index("c")
        s_ = lax.axis_index("s")
        wid = s_ * 2 + c
        sbase = s_ * STRIPE
        pltpu.sync_copy(zrows.at[pl.ds(sbase, STRIPE)],
                        accum.at[pl.ds(sbase, STRIPE)])
        plsc.subcore_barrier()

        def chunk(t, _):
            base = (wid * CPW + t) * CHUNK
            pltpu.sync_copy(idx.at[pl.ds(base, CHUNK)], idxbuf)
            pltpu.sync_copy(msg.at[pl.ds(base, CHUNK)], rowbuf)
            pltpu.sync_copy(rowbuf, accum.at[idxbuf], add=True)
            return _

        lax.fori_loop(0, CPW, chunk, None)
        plsc.subcore_barrier()
        pltpu.sync_copy(accum.at[pl.ds(sbase, STRIPE)],
                        part.at[pl.ds(c * NP_ + sbase, STRIPE)])

    return _scatter_k


def _fcol(a, q):
    if q < 2:
        return ("gs", 2 * a + q)
    if q < 4:
        return ("gd", 2 * a + q - 2)
    return ("ef", a)


def _tcol(a, q):
    if q < 2:
        return ("tab", 2 * a + q)
    return ("fc", 2 * a + q - 2)


def _gcol(src_col):
    src, col = src_col
    return {"gs": 0, "gd": TW, "ef": 2 * TW}[src] + col


def _ncol(src_col):
    src, col = src_col
    return {"tab": 0, "fc": TW}[src] + col


GW = 2 * TW + 3 + 4
NW_ = TW + MF


def _edge_consts():
    SA = np.zeros((GW, 75), np.float32)
    SB = np.zeros((GW, 75), np.float32)
    for j in range(3):
        for i in range(5):
            for k in range(5):
                t = j * 25 + i * 5 + k
                SA[_gcol(_fcol(j, i)), t] = 1.0
                SB[_gcol(_fcol(j, k)), t] = 1.0
    EC = np.zeros((75, 25), np.float32)
    for j in range(3):
        for i in range(5):
            for k in range(5):
                EC[j * 25 + i * 5 + k, i * 5 + k] = 1.0
    SM = np.zeros((GW, 30), np.float32)
    MB = np.zeros((42, 30), np.float32)
    MC = np.zeros((30, MF), np.float32)
    for i in range(3):
        for k in range(2):
            for j in range(5):
                u = i * 10 + k * 5 + j
                SM[_gcol(_fcol(i, j)), u] = 1.0
                MB[2 * j + k, u] = 1.0
                MC[u, i * 2 + k] = 1.0
    CNT = np.zeros((1, MF), np.float32)
    CNT[0, 6] = 1.0
    return (SA, SB, EC, SM, MB, MC, CNT)


def _node_consts():
    NSA = np.zeros((NW_, 48), np.float32)
    NSB = np.zeros((NW_, 48), np.float32)
    NC = np.zeros((48, 16), np.float32)
    for j in range(3):
        for i in range(4):
            for k in range(4):
                t = j * 16 + i * 4 + k
                NSA[_ncol(_tcol(j, i)), t] = 1.0
                NSB[_ncol(_tcol(j, k)), t] = 1.0
                NC[t, i * 4 + k] = 1.0
    NPA = np.zeros((NW_, 24), np.float32)
    Q = np.zeros((40, 24), np.float32)
    R = np.zeros((24, TW), np.float32)
    for i in range(3):
        for k in range(2):
            for j in range(4):
                u = i * 8 + k * 4 + j
                NPA[_ncol(_tcol(i, j)), u] = 1.0
                Q[2 * j + k, u] = 1.0
                R[u, i * 2 + k] = 1.0
    S = np.zeros((40, TW), np.float32)
    for m in range(SD):
        S[8 + m, 16 + m] = 1.0
    return (NSA, NSB, NC, NPA, Q, R, S)


_EDGE_C = _edge_consts()
_NODE_C = _node_consts()


def _silu(x):
    return jax.nn.silu(x)


def _mmd(a, b):
    return jnp.dot(a, b, preferred_element_type=jnp.float32)


def _mmh(a, b):
    ah = a.astype(jnp.bfloat16).astype(jnp.float32)
    return _mmd(ah, b) + _mmd(a - ah, b)


def _edge_body(gs_ref, gd_ref, ef_ref, es_ref,
               w1g, w1cat, b1, w2, b2, w3, b3,
               sa, sb, ec, sm, mb, mc, cnt,
               msgf_ref, msgs_ref):
    g = jnp.concatenate(
        [gs_ref[...], gd_ref[...], ef_ref[...], es_ref[...]], axis=1)
    ag = _mmh(g, sa[...])
    bg = _mmh(g, sb[...])
    gram = _mmh(ag * bg, ec[...])
    ss = jnp.sum(gram * gram, axis=1, keepdims=True)
    gram = gram / jnp.maximum(jnp.sqrt(ss), 1e-12)
    h = _silu(_mmd(gram, w1g[...]) + _mmd(g, w1cat[...]) + b1[...])
    h = _silu(_mmd(h, w2[...]) + b2[...])
    cc = _mmd(h, w3[...]) + b3[...]
    am = _mmh(g, sm[...])
    bm = _mmh(cc, mb[...])
    msgf_ref[...] = _mmh(am * bm, mc[...]) + cnt[...]
    msgs_ref[...] = cc[:, 10:42]


def _edge_call(gsrc, gdst, efp, esp, nw, ecst):
    nb = EP_ // BE
    full = lambda a: pl.BlockSpec(a.shape, lambda i: (0,) * a.ndim)
    return pl.pallas_call(
        _edge_body,
        grid=(nb,),
        in_specs=[
            pl.BlockSpec((BE, TW), lambda i: (i, 0)),
            pl.BlockSpec((BE, TW), lambda i: (i, 0)),
            pl.BlockSpec((BE, 3), lambda i: (i, 0)),
            pl.BlockSpec((BE, 4), lambda i: (i, 0)),
        ] + [full(a) for a in nw] + [full(a) for a in ecst],
        out_specs=[
            pl.BlockSpec((BE, MF), lambda i: (i, 0)),
            pl.BlockSpec((BE, SD), lambda i: (i, 0)),
        ],
        out_shape=[
            jax.ShapeDtypeStruct((EP_, MF), jnp.float32),
            jax.ShapeDtypeStruct((EP_, SD), jnp.float32),
        ],
    )(gsrc, gdst, efp, esp, *nw, *ecst)


def _node_body(tab_ref, fp0, fp1, sp0, sp1,
               w1g, w1cat, w1sc, b1, w2, b2, w3, b3,
               nsa, nsb, nc, npa, q, r, s_, out_ref):
    fsum = fp0[...] + fp1[...]
    inv = 1.0 / jnp.maximum(fsum[:, 6:7], 1.0)
    ssum = (sp0[...] + sp1[...]) * inv
    t = jnp.concatenate([tab_ref[...], fsum * inv], axis=1)
    ag = _mmh(t, nsa[...])
    bg = _mmh(t, nsb[...])
    gram = _mmh(ag * bg, nc[...])
    ss = jnp.sum(gram * gram, axis=1, keepdims=True)
    gram = gram / jnp.maximum(jnp.sqrt(ss), 1e-12)
    h = _silu(_mmd(gram, w1g[...]) + _mmd(t, w1cat[...])
              + _mmd(ssum, w1sc[...]) + b1[...])
    h = _silu(_mmd(h, w2[...]) + b2[...])
    tc = _mmd(h, w3[...]) + b3[...]
    a2 = _mmh(t, npa[...])
    b2_ = _mmh(tc, q[...])
    out_ref[...] = _mmh(a2 * b2_, r[...]) + _mmh(tc, s_[...])


def _node_call(tab, fpart, spart, sw, ncst):
    nb = NP_ // BN
    off = NP_ // BN
    full = lambda a: pl.BlockSpec(a.shape, lambda i: (0,) * a.ndim)
    return pl.pallas_call(
        _node_body,
        grid=(nb,),
        in_specs=[
            pl.BlockSpec((BN, TW), lambda i: (i, 0)),
            pl.BlockSpec((BN, MF), lambda i: (i, 0)),
            pl.BlockSpec((BN, MF), lambda i: (i + off, 0)),
            pl.BlockSpec((BN, SD), lambda i: (i, 0)),
            pl.BlockSpec((BN, SD), lambda i: (i + off, 0)),
        ] + [full(a) for a in sw] + [full(a) for a in ncst],
        out_specs=pl.BlockSpec((BN, TW), lambda i: (i, 0)),
        out_shape=jax.ShapeDtypeStruct((NP_, TW), jnp.float32),
    )(tab, fpart, fpart, spart, spart, *sw, *ncst)


def _edge_weights(p):
    w1 = p["W1"]
    w1cat = jnp.concatenate([
        jnp.zeros((16, HD), jnp.float32), w1[25:57],
        jnp.zeros((16, HD), jnp.float32), w1[57:89],
        jnp.zeros((3, HD), jnp.float32), w1[89:93],
    ], axis=0)
    return (w1[:25], w1cat,
            p["b1"].reshape(1, -1), p["W2"], p["b2"].reshape(1, -1),
            p["W3"], p["b3"].reshape(1, -1))


def _node_weights(p):
    w1 = p["W1"]
    w1cat = jnp.concatenate([
        jnp.zeros((16, HD), jnp.float32), w1[16:48],
        jnp.zeros((MF, HD), jnp.float32),
    ], axis=0)
    return (w1[:16], w1cat, w1[48:80],
            p["b1"].reshape(1, -1), p["W2"], p["b2"].reshape(1, -1),
            p["W3"], p["b3"].reshape(1, -1))


def kernel(f, s, edge_index, edge_f, edge_s, net, self_net):
    ei = edge_index.astype(jnp.int32)
    pad = jnp.full((EP_ - E,), N, jnp.int32)
    srcp = jnp.concatenate([ei[0], pad])
    dstp = jnp.concatenate([ei[1], pad])
    efp = jnp.pad(edge_f.reshape(E, 3), ((0, EP_ - E), (0, 0)))
    esp = jnp.pad(edge_s, ((0, EP_ - E), (0, 0)))
    tab = jnp.concatenate([
        jnp.pad(f.reshape(N, 6), ((0, NP_ - N), (0, 0))),
        jnp.zeros((NP_, 10), jnp.float32),
        jnp.pad(s, ((0, NP_ - N), (0, 0))),
    ], axis=1)
    zf = jnp.zeros((NP_, MF), jnp.float32)
    zs = jnp.zeros((NP_, SD), jnp.float32)
    nw = _edge_weights(net)
    sw = _node_weights(self_net)
    ecst = tuple(jnp.asarray(m) for m in _EDGE_C)
    ncst = tuple(jnp.asarray(m) for m in _NODE_C)
    gather_k = _build_gather()
    scatter_f = _build_scatter(MF)
    scatter_s = _build_scatter(SD)
    for _ in range(PSTEP):
        gsrc, gdst = gather_k(tab, srcp, dstp)
        msgf, msgs = _edge_call(gsrc, gdst, efp, esp, nw, ecst)
        fpart = scatter_f(msgf, srcp, zf)
        spart = scatter_s(msgs, srcp, zs)
        tab = _node_call(tab, fpart, spart, sw, ncst)
    return tab[:N, :6].reshape(N, 3, FD), tab[:N, 16:48]


# speedup vs baseline: 1.8422x; 1.0127x over previous
import functools

import jax
import jax.numpy as jnp
import numpy as np
from jax import lax
from jax.experimental import pallas as pl
from jax.experimental.pallas import tpu as pltpu
from jax.experimental.pallas import tpu_sc as plsc

N = 50000
E = 800000
FD = 2
SD = 32
HD = 32
PSTEP = 4

NP_ = 50176
EP_ = 819200
TW = 48
MF = 16
CHUNK = 128
NWORK = 32
CPW = EP_ // (NWORK * CHUNK)
STRIPE = NP_ // 16

BE = 4096
BN = 1024

@functools.lru_cache(maxsize=None)
def _build_gather():
    mesh = plsc.VectorSubcoreMesh(core_axis_name="c", subcore_axis_name="s")

    @functools.partial(
        pl.kernel,
        out_type=(
            jax.ShapeDtypeStruct((EP_, TW), jnp.float32),
            jax.ShapeDtypeStruct((EP_, TW), jnp.float32),
        ),
        scratch_types=[
            pltpu.VMEM((CHUNK,), jnp.int32),
            pltpu.VMEM((CHUNK, TW), jnp.float32),
            pltpu.SemaphoreType.DMA,
        ],
        mesh=mesh,
        compiler_params=pltpu.CompilerParams(use_tc_tiling_on_sc=False),
    )
    def _gather_k(tab, srcp, dstp, gsrc, gdst, idxbuf, rowbuf, sem):
        wid = lax.axis_index("s") * 2 + lax.axis_index("c")

        def chunk(t, _):
            base = (wid * CPW + t) * CHUNK
            pltpu.sync_copy(srcp.at[pl.ds(base, CHUNK)], idxbuf)
            pltpu.async_copy(tab.at[idxbuf], rowbuf, sem).wait()
            pltpu.sync_copy(rowbuf, gsrc.at[pl.ds(base, CHUNK)])
            pltpu.sync_copy(dstp.at[pl.ds(base, CHUNK)], idxbuf)
            pltpu.async_copy(tab.at[idxbuf], rowbuf, sem).wait()
            pltpu.sync_copy(rowbuf, gdst.at[pl.ds(base, CHUNK)])
            return _

        lax.fori_loop(0, CPW, chunk, None)

    return _gather_k


@functools.lru_cache(maxsize=None)
def _build_scatter(w):
    mesh = plsc.VectorSubcoreMesh(core_axis_name="c", subcore_axis_name="s")

    @functools.partial(
        pl.kernel,
        out_type=jax.ShapeDtypeStruct((2 * NP_, w), jnp.float32),
        scratch_types=[
            pltpu.VMEM((CHUNK,), jnp.int32),
            pltpu.VMEM((CHUNK, w), jnp.float32),
            pltpu.VMEM_SHARED((NP_, w), jnp.float32),
            pltpu.SemaphoreType.DMA,
        ],
        mesh=mesh,
        compiler_params=pltpu.CompilerParams(use_tc_tiling_on_sc=False),
    )
    def _scatter_k(msg, idx, zrows, part, idxbuf, rowbuf, accum, sem):
        c = lax.axis_index("c")
        s_ = lax.axis_index("s")
        wid = s_ * 2 + c
        sbase = s_ * STRIPE
        pltpu.sync_copy(zrows.at[pl.ds(sbase, STRIPE)],
                        accum.at[pl.ds(sbase, STRIPE)])
        plsc.subcore_barrier()

        def chunk(t, _):
            base = (wid * CPW + t) * CHUNK
            pltpu.sync_copy(idx.at[pl.ds(base, CHUNK)], idxbuf)
            pltpu.sync_copy(msg.at[pl.ds(base, CHUNK)], rowbuf)
            pltpu.sync_copy(rowbuf, accum.at[idxbuf], add=True)
            return _

        lax.fori_loop(0, CPW, chunk, None)
        plsc.subcore_barrier()
        pltpu.sync_copy(accum.at[pl.ds(sbase, STRIPE)],
                        part.at[pl.ds(c * NP_ + sbase, STRIPE)])

    return _scatter_k


def _fcol(a, q):
    if q < 2:
        return ("gs", 2 * a + q)
    if q < 4:
        return ("gd", 2 * a + q - 2)
    return ("ef", a)


def _tcol(a, q):
    if q < 2:
        return ("tab", 2 * a + q)
    return ("fc", 2 * a + q - 2)


def _gcol(src_col):
    src, col = src_col
    return {"gs": 0, "gd": TW, "ef": 2 * TW}[src] + col


def _ncol(src_col):
    src, col = src_col
    return {"tab": 0, "fc": TW}[src] + col


GW = 2 * TW + 3 + 4
NW_ = TW + MF


def _edge_consts():
    SA = np.zeros((GW, 75), np.float32)
    SB = np.zeros((GW, 75), np.float32)
    for j in range(3):
        for i in range(5):
            for k in range(5):
                t = j * 25 + i * 5 + k
                SA[_gcol(_fcol(j, i)), t] = 1.0
                SB[_gcol(_fcol(j, k)), t] = 1.0
    EC = np.zeros((75, 25), np.float32)
    for j in range(3):
        for i in range(5):
            for k in range(5):
                EC[j * 25 + i * 5 + k, i * 5 + k] = 1.0
    SM = np.zeros((GW, 30), np.float32)
    MB = np.zeros((42, 30), np.float32)
    MC = np.zeros((30, MF), np.float32)
    for i in range(3):
        for k in range(2):
            for j in range(5):
                u = i * 10 + k * 5 + j
                SM[_gcol(_fcol(i, j)), u] = 1.0
                MB[2 * j + k, u] = 1.0
                MC[u, i * 2 + k] = 1.0
    CNT = np.zeros((1, MF), np.float32)
    CNT[0, 6] = 1.0
    return (SA, SB, EC, SM, MB, MC, CNT)


def _node_consts():
    NSA = np.zeros((NW_, 48), np.float32)
    NSB = np.zeros((NW_, 48), np.float32)
    NC = np.zeros((48, 16), np.float32)
    for j in range(3):
        for i in range(4):
            for k in range(4):
                t = j * 16 + i * 4 + k
                NSA[_ncol(_tcol(j, i)), t] = 1.0
                NSB[_ncol(_tcol(j, k)), t] = 1.0
                NC[t, i * 4 + k] = 1.0
    NPA = np.zeros((NW_, 24), np.float32)
    Q = np.zeros((40, 24), np.float32)
    R = np.zeros((24, TW), np.float32)
    for i in range(3):
        for k in range(2):
            for j in range(4):
                u = i * 8 + k * 4 + j
                NPA[_ncol(_tcol(i, j)), u] = 1.0
                Q[2 * j + k, u] = 1.0
                R[u, i * 2 + k] = 1.0
    S = np.zeros((40, TW), np.float32)
    for m in range(SD):
        S[8 + m, 16 + m] = 1.0
    return (NSA, NSB, NC, NPA, Q, R, S)


_EDGE_C = _edge_consts()
_NODE_C = _node_consts()


def _silu(x):
    return jax.nn.silu(x)


def _mmd(a, b):
    return jnp.dot(a, b, preferred_element_type=jnp.float32)


def _mmh(a, b):
    ah = a.astype(jnp.bfloat16).astype(jnp.float32)
    return _mmd(ah, b) + _mmd(a - ah, b)


def _edge_body(gs_ref, gd_ref, ef_ref, es_ref,
               w1g, w1cat, b1, w2, b2, w3, b3,
               sa, sb, ec, sm, mb, mc, cnt,
               msgf_ref, msgs_ref):
    g = jnp.concatenate(
        [gs_ref[...], gd_ref[...], ef_ref[...], es_ref[...]], axis=1)
    ag = _mmh(g, sa[...])
    bg = _mmh(g, sb[...])
    gram = _mmh(ag * bg, ec[...])
    ss = jnp.sum(gram * gram, axis=1, keepdims=True)
    gram = gram / jnp.maximum(jnp.sqrt(ss), 1e-12)
    h = _silu(_mmd(gram, w1g[...]) + _mmd(g, w1cat[...]) + b1[...])
    h = _silu(_mmd(h, w2[...]) + b2[...])
    cc = _mmd(h, w3[...]) + b3[...]
    am = _mmh(g, sm[...])
    bm = _mmh(cc, mb[...])
    msgf_ref[...] = _mmh(am * bm, mc[...]) + cnt[...]
    msgs_ref[...] = cc[:, 10:42]


def _edge_call(gsrc, gdst, efp, esp, nw, ecst):
    nb = EP_ // BE
    full = lambda a: pl.BlockSpec(a.shape, lambda i: (0,) * a.ndim)
    return pl.pallas_call(
        _edge_body,
        grid=(nb,),
        in_specs=[
            pl.BlockSpec((BE, TW), lambda i: (i, 0)),
            pl.BlockSpec((BE, TW), lambda i: (i, 0)),
            pl.BlockSpec((BE, 3), lambda i: (i, 0)),
            pl.BlockSpec((BE, 4), lambda i: (i, 0)),
        ] + [full(a) for a in nw] + [full(a) for a in ecst],
        out_specs=[
            pl.BlockSpec((BE, MF), lambda i: (i, 0)),
            pl.BlockSpec((BE, SD), lambda i: (i, 0)),
        ],
        out_shape=[
            jax.ShapeDtypeStruct((EP_, MF), jnp.float32),
            jax.ShapeDtypeStruct((EP_, SD), jnp.float32),
        ],
    )(gsrc, gdst, efp, esp, *nw, *ecst)


def _node_body(tab_ref, fp0, fp1, sp0, sp1,
               w1g, w1cat, w1sc, b1, w2, b2, w3, b3,
               nsa, nsb, nc, npa, q, r, s_, out_ref):
    fsum = fp0[...] + fp1[...]
    inv = 1.0 / jnp.maximum(fsum[:, 6:7], 1.0)
    ssum = (sp0[...] + sp1[...]) * inv
    t = jnp.concatenate([tab_ref[...], fsum * inv], axis=1)
    ag = _mmh(t, nsa[...])
    bg = _mmh(t, nsb[...])
    gram = _mmh(ag * bg, nc[...])
    ss = jnp.sum(gram * gram, axis=1, keepdims=True)
    gram = gram / jnp.maximum(jnp.sqrt(ss), 1e-12)
    h = _silu(_mmd(gram, w1g[...]) + _mmd(t, w1cat[...])
              + _mmd(ssum, w1sc[...]) + b1[...])
    h = _silu(_mmd(h, w2[...]) + b2[...])
    tc = _mmd(h, w3[...]) + b3[...]
    a2 = _mmh(t, npa[...])
    b2_ = _mmh(tc, q[...])
    out_ref[...] = _mmh(a2 * b2_, r[...]) + _mmh(tc, s_[...])


def _node_call(tab, fpart, spart, sw, ncst):
    nb = NP_ // BN
    off = NP_ // BN
    full = lambda a: pl.BlockSpec(a.shape, lambda i: (0,) * a.ndim)
    return pl.pallas_call(
        _node_body,
        grid=(nb,),
        in_specs=[
            pl.BlockSpec((BN, TW), lambda i: (i, 0)),
            pl.BlockSpec((BN, MF), lambda i: (i, 0)),
            pl.BlockSpec((BN, MF), lambda i: (i + off, 0)),
            pl.BlockSpec((BN, SD), lambda i: (i, 0)),
            pl.BlockSpec((BN, SD), lambda i: (i + off, 0)),
        ] + [full(a) for a in sw] + [full(a) for a in ncst],
        out_specs=pl.BlockSpec((BN, TW), lambda i: (i, 0)),
        out_shape=jax.ShapeDtypeStruct((NP_, TW), jnp.float32),
    )(tab, fpart, fpart, spart, spart, *sw, *ncst)


def _edge_weights(p):
    w1 = p["W1"]
    w1cat = jnp.concatenate([
        jnp.zeros((16, HD), jnp.float32), w1[25:57],
        jnp.zeros((16, HD), jnp.float32), w1[57:89],
        jnp.zeros((3, HD), jnp.float32), w1[89:93],
    ], axis=0)
    return (w1[:25], w1cat,
            p["b1"].reshape(1, -1), p["W2"], p["b2"].reshape(1, -1),
            p["W3"], p["b3"].reshape(1, -1))


def _node_weights(p):
    w1 = p["W1"]
    w1cat = jnp.concatenate([
        jnp.zeros((16, HD), jnp.float32), w1[16:48],
        jnp.zeros((MF, HD), jnp.float32),
    ], axis=0)
    return (w1[:16], w1cat, w1[48:80],
            p["b1"].reshape(1, -1), p["W2"], p["b2"].reshape(1, -1),
            p["W3"], p["b3"].reshape(1, -1))


def kernel(f, s, edge_index, edge_f, edge_s, net, self_net):
    ei = edge_index.astype(jnp.int32)
    pad = jnp.full((EP_ - E,), N, jnp.int32)
    srcp = jnp.concatenate([ei[0], pad])
    dstp = jnp.concatenate([ei[1], pad])
    efp = jnp.pad(edge_f.reshape(E, 3), ((0, EP_ - E), (0, 0)))
    esp = jnp.pad(edge_s, ((0, EP_ - E), (0, 0)))
    tab = jnp.concatenate([
        jnp.pad(f.reshape(N, 6), ((0, NP_ - N), (0, 0))),
        jnp.zeros((NP_, 10), jnp.float32),
        jnp.pad(s, ((0, NP_ - N), (0, 0))),
    ], axis=1)
    zf = jnp.zeros((NP_, MF), jnp.float32)
    zs = jnp.zeros((NP_, SD), jnp.float32)
    nw = _edge_weights(net)
    sw = _node_weights(self_net)
    ecst = tuple(jnp.asarray(m) for m in _EDGE_C)
    ncst = tuple(jnp.asarray(m) for m in _NODE_C)
    gather_k = _build_gather()
    scatter_f = _build_scatter(MF)
    scatter_s = _build_scatter(SD)
    for _ in range(PSTEP):
        gsrc, gdst = gather_k(tab, srcp, dstp)
        msgf, msgs = _edge_call(gsrc, gdst, efp, esp, nw, ecst)
        fpart = scatter_f(msgf, srcp, zf)
        spart = scatter_s(msgs, srcp, zs)
        tab = _node_call(tab, fpart, spart, sw, ncst)
    return tab[:N, :6].reshape(N, 3, FD), tab[:N, 16:48]


# speedup vs baseline: 1.9086x; 1.0361x over previous
import functools

import jax
import jax.numpy as jnp
import numpy as np
from jax import lax
from jax.experimental import pallas as pl
from jax.experimental.pallas import tpu as pltpu
from jax.experimental.pallas import tpu_sc as plsc

N = 50000
E = 800000
FD = 2
SD = 32
HD = 32
PSTEP = 4

NP_ = 50176
EP_ = 819200
TW = 48
MF = 16
CHUNK = 128
NWORK = 32
CPW = EP_ // (NWORK * CHUNK)
STRIPE = NP_ // 16

BE = 4096
BN = 1024

@functools.lru_cache(maxsize=None)
def _build_gather():
    mesh = plsc.VectorSubcoreMesh(core_axis_name="c", subcore_axis_name="s")

    @functools.partial(
        pl.kernel,
        out_type=(
            jax.ShapeDtypeStruct((EP_, TW), jnp.float32),
            jax.ShapeDtypeStruct((EP_, TW), jnp.float32),
        ),
        scratch_types=[
            pltpu.VMEM((CHUNK,), jnp.int32),
            pltpu.VMEM((CHUNK,), jnp.int32),
            pltpu.VMEM((CHUNK, TW), jnp.float32),
            pltpu.VMEM((CHUNK, TW), jnp.float32),
            pltpu.SemaphoreType.DMA,
            pltpu.SemaphoreType.DMA,
        ],
        mesh=mesh,
        compiler_params=pltpu.CompilerParams(use_tc_tiling_on_sc=False),
    )
    def _gather_k(tab, srcp, dstp, gsrc, gdst,
                  idxbuf, idxbuf2, rowbuf, rowbuf2, sem, sem2):
        wid = lax.axis_index("s") * 2 + lax.axis_index("c")

        def chunk(t, _):
            base = (wid * CPW + t) * CHUNK
            pltpu.sync_copy(srcp.at[pl.ds(base, CHUNK)], idxbuf)
            cp1 = pltpu.async_copy(tab.at[idxbuf], rowbuf, sem)
            pltpu.sync_copy(dstp.at[pl.ds(base, CHUNK)], idxbuf2)
            cp2 = pltpu.async_copy(tab.at[idxbuf2], rowbuf2, sem2)
            cp1.wait()
            pltpu.sync_copy(rowbuf, gsrc.at[pl.ds(base, CHUNK)])
            cp2.wait()
            pltpu.sync_copy(rowbuf2, gdst.at[pl.ds(base, CHUNK)])
            return _

        lax.fori_loop(0, CPW, chunk, None)

    return _gather_k


@functools.lru_cache(maxsize=None)
def _build_scatter(w):
    mesh = plsc.VectorSubcoreMesh(core_axis_name="c", subcore_axis_name="s")

    @functools.partial(
        pl.kernel,
        out_type=jax.ShapeDtypeStruct((2 * NP_, w), jnp.float32),
        scratch_types=[
            pltpu.VMEM((CHUNK,), jnp.int32),
            pltpu.VMEM((CHUNK, w), jnp.float32),
            pltpu.VMEM_SHARED((NP_, w), jnp.float32),
            pltpu.SemaphoreType.DMA,
        ],
        mesh=mesh,
        compiler_params=pltpu.CompilerParams(use_tc_tiling_on_sc=False),
    )
    def _scatter_k(msg, idx, zrows, part, idxbuf, rowbuf, accum, sem):
        c = lax.axis_index("c")
        s_ = lax.axis_index("s")
        wid = s_ * 2 + c
        sbase = s_ * STRIPE
        pltpu.sync_copy(zrows.at[pl.ds(sbase, STRIPE)],
                        accum.at[pl.ds(sbase, STRIPE)])
        plsc.subcore_barrier()

        def chunk(t, _):
            base = (wid * CPW + t) * CHUNK
            pltpu.sync_copy(idx.at[pl.ds(base, CHUNK)], idxbuf)
            pltpu.sync_copy(msg.at[pl.ds(base, CHUNK)], rowbuf)
            pltpu.sync_copy(rowbuf, accum.at[idxbuf], add=True)
            return _

        lax.fori_loop(0, CPW, chunk, None)
        plsc.subcore_barrier()
        pltpu.sync_copy(accum.at[pl.ds(sbase, STRIPE)],
                        part.at[pl.ds(c * NP_ + sbase, STRIPE)])

    return _scatter_k


def _fcol(a, q):
    if q < 2:
        return ("gs", 2 * a + q)
    if q < 4:
        return ("gd", 2 * a + q - 2)
    return ("ef", a)


def _tcol(a, q):
    if q < 2:
        return ("tab", 2 * a + q)
    return ("fc", 2 * a + q - 2)


def _gcol(src_col):
    src, col = src_col
    return {"gs": 0, "gd": TW, "ef": 2 * TW}[src] + col


def _ncol(src_col):
    src, col = src_col
    return {"tab": 0, "fc": TW}[src] + col


GW = 2 * TW + 3 + 4
NW_ = TW + MF


def _edge_consts():
    SA = np.zeros((GW, 75), np.float32)
    SB = np.zeros((GW, 75), np.float32)
    for j in range(3):
        for i in range(5):
            for k in range(5):
                t = j * 25 + i * 5 + k
                SA[_gcol(_fcol(j, i)), t] = 1.0
                SB[_gcol(_fcol(j, k)), t] = 1.0
    EC = np.zeros((75, 25), np.float32)
    for j in range(3):
        for i in range(5):
            for k in range(5):
                EC[j * 25 + i * 5 + k, i * 5 + k] = 1.0
    SM = np.zeros((GW, 30), np.float32)
    MB = np.zeros((42, 30), np.float32)
    MC = np.zeros((30, MF), np.float32)
    for i in range(3):
        for k in range(2):
            for j in range(5):
                u = i * 10 + k * 5 + j
                SM[_gcol(_fcol(i, j)), u] = 1.0
                MB[2 * j + k, u] = 1.0
                MC[u, i * 2 + k] = 1.0
    CNT = np.zeros((1, MF), np.float32)
    CNT[0, 6] = 1.0
    return (SA, SB, EC, SM, MB, MC, CNT)


def _node_consts():
    NSA = np.zeros((NW_, 48), np.float32)
    NSB = np.zeros((NW_, 48), np.float32)
    NC = np.zeros((48, 16), np.float32)
    for j in range(3):
        for i in range(4):
            for k in range(4):
                t = j * 16 + i * 4 + k
                NSA[_ncol(_tcol(j, i)), t] = 1.0
                NSB[_ncol(_tcol(j, k)), t] = 1.0
                NC[t, i * 4 + k] = 1.0
    NPA = np.zeros((NW_, 24), np.float32)
    Q = np.zeros((40, 24), np.float32)
    R = np.zeros((24, TW), np.float32)
    for i in range(3):
        for k in range(2):
            for j in range(4):
                u = i * 8 + k * 4 + j
                NPA[_ncol(_tcol(i, j)), u] = 1.0
                Q[2 * j + k, u] = 1.0
                R[u, i * 2 + k] = 1.0
    S = np.zeros((40, TW), np.float32)
    for m in range(SD):
        S[8 + m, 16 + m] = 1.0
    return (NSA, NSB, NC, NPA, Q, R, S)


_EDGE_C = _edge_consts()
_NODE_C = _node_consts()


def _silu(x):
    return jax.nn.silu(x)


def _mmd(a, b):
    return jnp.dot(a, b, preferred_element_type=jnp.float32)


def _mmh(a, b):
    ah = a.astype(jnp.bfloat16).astype(jnp.float32)
    return _mmd(ah, b) + _mmd(a - ah, b)


def _edge_body(gs_ref, gd_ref, ef_ref, es_ref,
               w1g, w1cat, b1, w2, b2, w3, b3,
               sa, sb, ec, sm, mb, mc, cnt,
               msgf_ref, msgs_ref):
    g = jnp.concatenate(
        [gs_ref[...], gd_ref[...], ef_ref[...], es_ref[...]], axis=1)
    ag = _mmh(g, sa[...])
    bg = _mmh(g, sb[...])
    gram = _mmh(ag * bg, ec[...])
    ss = jnp.sum(gram * gram, axis=1, keepdims=True)
    gram = gram / jnp.maximum(jnp.sqrt(ss), 1e-12)
    h = _silu(_mmd(gram, w1g[...]) + _mmd(g, w1cat[...]) + b1[...])
    h = _silu(_mmd(h, w2[...]) + b2[...])
    cc = _mmd(h, w3[...]) + b3[...]
    am = _mmh(g, sm[...])
    bm = _mmh(cc, mb[...])
    msgf_ref[...] = _mmh(am * bm, mc[...]) + cnt[...]
    msgs_ref[...] = cc[:, 10:42]


def _edge_call(gsrc, gdst, efp, esp, nw, ecst):
    nb = EP_ // BE
    full = lambda a: pl.BlockSpec(a.shape, lambda i: (0,) * a.ndim)
    return pl.pallas_call(
        _edge_body,
        grid=(nb,),
        in_specs=[
            pl.BlockSpec((BE, TW), lambda i: (i, 0)),
            pl.BlockSpec((BE, TW), lambda i: (i, 0)),
            pl.BlockSpec((BE, 3), lambda i: (i, 0)),
            pl.BlockSpec((BE, 4), lambda i: (i, 0)),
        ] + [full(a) for a in nw] + [full(a) for a in ecst],
        out_specs=[
            pl.BlockSpec((BE, MF), lambda i: (i, 0)),
            pl.BlockSpec((BE, SD), lambda i: (i, 0)),
        ],
        out_shape=[
            jax.ShapeDtypeStruct((EP_, MF), jnp.float32),
            jax.ShapeDtypeStruct((EP_, SD), jnp.float32),
        ],
    )(gsrc, gdst, efp, esp, *nw, *ecst)


def _node_body(tab_ref, fp0, fp1, sp0, sp1,
               w1g, w1cat, w1sc, b1, w2, b2, w3, b3,
               nsa, nsb, nc, npa, q, r, s_, out_ref):
    fsum = fp0[...] + fp1[...]
    inv = 1.0 / jnp.maximum(fsum[:, 6:7], 1.0)
    ssum = (sp0[...] + sp1[...]) * inv
    t = jnp.concatenate([tab_ref[...], fsum * inv], axis=1)
    ag = _mmh(t, nsa[...])
    bg = _mmh(t, nsb[...])
    gram = _mmh(ag * bg, nc[...])
    ss = jnp.sum(gram * gram, axis=1, keepdims=True)
    gram = gram / jnp.maximum(jnp.sqrt(ss), 1e-12)
    h = _silu(_mmd(gram, w1g[...]) + _mmd(t, w1cat[...])
              + _mmd(ssum, w1sc[...]) + b1[...])
    h = _silu(_mmd(h, w2[...]) + b2[...])
    tc = _mmd(h, w3[...]) + b3[...]
    a2 = _mmh(t, npa[...])
    b2_ = _mmh(tc, q[...])
    out_ref[...] = _mmh(a2 * b2_, r[...]) + _mmh(tc, s_[...])


def _node_call(tab, fpart, spart, sw, ncst):
    nb = NP_ // BN
    off = NP_ // BN
    full = lambda a: pl.BlockSpec(a.shape, lambda i: (0,) * a.ndim)
    return pl.pallas_call(
        _node_body,
        grid=(nb,),
        in_specs=[
            pl.BlockSpec((BN, TW), lambda i: (i, 0)),
            pl.BlockSpec((BN, MF), lambda i: (i, 0)),
            pl.BlockSpec((BN, MF), lambda i: (i + off, 0)),
            pl.BlockSpec((BN, SD), lambda i: (i, 0)),
            pl.BlockSpec((BN, SD), lambda i: (i + off, 0)),
        ] + [full(a) for a in sw] + [full(a) for a in ncst],
        out_specs=pl.BlockSpec((BN, TW), lambda i: (i, 0)),
        out_shape=jax.ShapeDtypeStruct((NP_, TW), jnp.float32),
    )(tab, fpart, fpart, spart, spart, *sw, *ncst)


def _edge_weights(p):
    w1 = p["W1"]
    w1cat = jnp.concatenate([
        jnp.zeros((16, HD), jnp.float32), w1[25:57],
        jnp.zeros((16, HD), jnp.float32), w1[57:89],
        jnp.zeros((3, HD), jnp.float32), w1[89:93],
    ], axis=0)
    return (w1[:25], w1cat,
            p["b1"].reshape(1, -1), p["W2"], p["b2"].reshape(1, -1),
            p["W3"], p["b3"].reshape(1, -1))


def _node_weights(p):
    w1 = p["W1"]
    w1cat = jnp.concatenate([
        jnp.zeros((16, HD), jnp.float32), w1[16:48],
        jnp.zeros((MF, HD), jnp.float32),
    ], axis=0)
    return (w1[:16], w1cat, w1[48:80],
            p["b1"].reshape(1, -1), p["W2"], p["b2"].reshape(1, -1),
            p["W3"], p["b3"].reshape(1, -1))


def kernel(f, s, edge_index, edge_f, edge_s, net, self_net):
    ei = edge_index.astype(jnp.int32)
    pad = jnp.full((EP_ - E,), N, jnp.int32)
    srcp = jnp.concatenate([ei[0], pad])
    dstp = jnp.concatenate([ei[1], pad])
    efp = jnp.pad(edge_f.reshape(E, 3), ((0, EP_ - E), (0, 0)))
    esp = jnp.pad(edge_s, ((0, EP_ - E), (0, 0)))
    tab = jnp.concatenate([
        jnp.pad(f.reshape(N, 6), ((0, NP_ - N), (0, 0))),
        jnp.zeros((NP_, 10), jnp.float32),
        jnp.pad(s, ((0, NP_ - N), (0, 0))),
    ], axis=1)
    zf = jnp.zeros((NP_, MF), jnp.float32)
    zs = jnp.zeros((NP_, SD), jnp.float32)
    nw = _edge_weights(net)
    sw = _node_weights(self_net)
    ecst = tuple(jnp.asarray(m) for m in _EDGE_C)
    ncst = tuple(jnp.asarray(m) for m in _NODE_C)
    gather_k = _build_gather()
    scatter_f = _build_scatter(MF)
    scatter_s = _build_scatter(SD)
    for _ in range(PSTEP):
        gsrc, gdst = gather_k(tab, srcp, dstp)
        msgf, msgs = _edge_call(gsrc, gdst, efp, esp, nw, ecst)
        fpart = scatter_f(msgf, srcp, zf)
        spart = scatter_s(msgs, srcp, zs)
        tab = _node_call(tab, fpart, spart, sw, ncst)
    return tab[:N, :6].reshape(N, 3, FD), tab[:N, 16:48]
